# Initial kernel scaffold; baseline (speedup 1.0000x reference)
import jax
import jax.numpy as jnp
from jax.experimental import pallas as pl


def kernel(x, edge_index, W1, b1, W2, b2, lw1, lb1, lw2, lb2, lw3, lb3):
    raise NotImplementedError("write your pallas kernel here")


# speedup vs baseline: 9.0960x; 9.0960x over previous
import functools

import jax
import jax.numpy as jnp
from jax import lax
from jax.experimental import pallas as pl
from jax.experimental.pallas import tpu as pltpu
from jax.experimental.pallas import tpu_sc as plsc

N_NODES = 10000
H = 128
NC = 2
NS = 16
NW = NC * NS
CH = 128
ACC_ROWS = 10240
RPT = ACC_ROWS // NS
DEGW = 16
BT = 1000

HALF = 40
K_FAST = 3 * HALF
K_SLOW = 1 * HALF
FAST_CID = 0
NCHUNK = K_FAST
E_PAD = NS * (K_FAST + K_SLOW) * CH

DCHUNK = 80
DHALF = DCHUNK // 2
E_PAD_DEG = NW * DCHUNK * CH


def _sc_deg(dstr):
    mesh = plsc.VectorSubcoreMesh(core_axis_name="c", subcore_axis_name="s")

    @functools.partial(
        pl.kernel,
        out_type=jax.ShapeDtypeStruct((NC, ACC_ROWS, H), jnp.float32),
        mesh=mesh,
        scratch_types=[
            pltpu.VMEM((DHALF, CH), jnp.int32),
            pltpu.VMEM((CH, H), jnp.float32),
            pltpu.VMEM_SHARED((ACC_ROWS, H), jnp.float32),
            pltpu.SemaphoreType.DMA,
        ],
    )
    def k(dst_hbm, out_hbm, dst_v, ones_v, acc, sem):
        cid = lax.axis_index("c")
        sid = lax.axis_index("s")
        wid = cid * NS + sid

        def fill(val):
            def body(i, carry):
                for j in range(H // 16):
                    ones_v[i, pl.ds(j * 16, 16)] = jnp.full((16,), val, jnp.float32)
                return carry
            lax.fori_loop(0, CH, body, 0)

        fill(0.0)
        base = sid * RPT
        for j in range(RPT // CH):
            pltpu.sync_copy(ones_v, acc.at[pl.ds(base + j * CH, CH)])
        fill(1.0)
        plsc.subcore_barrier()

        def body(g, carry):
            ds_ = [pltpu.async_copy(ones_v, acc.at[dst_v.at[g * 2 + b]], sem,
                                    add=True)
                   for b in range(2)]
            for d in ds_:
                d.wait()
            return carry

        for h in range(DCHUNK // DHALF):
            pltpu.sync_copy(dst_hbm.at[wid, pl.ds(h * DHALF, DHALF)], dst_v)
            lax.fori_loop(0, DHALF // 2, body, 0)
        plsc.subcore_barrier()
        pltpu.sync_copy(acc.at[pl.ds(base, RPT)], out_hbm.at[cid, pl.ds(base, RPT)])

    return k(dstr)


def _sc_agg(hs, srcr, dstr):
    mesh = plsc.VectorSubcoreMesh(core_axis_name="c", subcore_axis_name="s")

    @functools.partial(
        pl.kernel,
        out_type=jax.ShapeDtypeStruct((NC, ACC_ROWS, H), jnp.float32),
        mesh=mesh,
        scratch_types=[
            pltpu.VMEM((HALF, CH), jnp.int32),
            pltpu.VMEM((HALF, CH), jnp.int32),
            pltpu.VMEM((2, CH, H), jnp.float32),
            pltpu.VMEM_SHARED((ACC_ROWS, H), jnp.float32),
            pltpu.SemaphoreType.DMA,
            pltpu.SemaphoreType.DMA,
            pltpu.SemaphoreType.DMA,
            pltpu.SemaphoreType.DMA,
        ],
    )
    def k(hs_hbm, src_hbm, dst_hbm, out_hbm, src_v, dst_v, rows_v, acc,
          gsem0, gsem1, ssem0, ssem1):
        cid = lax.axis_index("c")
        sid = lax.axis_index("s")
        wid = cid * NS + sid
        nreal = jnp.where(cid == FAST_CID, K_FAST, K_SLOW)

        def zbody(i, carry):
            for j in range(H // 16):
                rows_v[0, i, pl.ds(j * 16, 16)] = jnp.zeros((16,), jnp.float32)
            return carry
        lax.fori_loop(0, CH, zbody, 0)
        base = sid * RPT
        for j in range(RPT // CH):
            pltpu.sync_copy(rows_v.at[0], acc.at[pl.ds(base + j * CH, CH)])
        plsc.subcore_barrier()

        def gfire(c, slot, sem):
            pltpu.async_copy(hs_hbm.at[src_v.at[c]], rows_v.at[slot], sem)

        def gwait(c, slot, sem):
            pltpu.make_async_copy(hs_hbm.at[src_v.at[c]], rows_v.at[slot],
                                  sem).wait()

        def sfire(c, slot, sem):
            pltpu.async_copy(rows_v.at[slot], acc.at[dst_v.at[c]], sem,
                             add=True)

        def swait(c, slot, sem):
            pltpu.make_async_copy(rows_v.at[slot], acc.at[dst_v.at[c]],
                                  sem).wait()

        NG = HALF // 2
        for h in range(NCHUNK // HALF):

            @pl.when(h * HALF < nreal)
            def _stage(h=h):
                pltpu.sync_copy(src_hbm.at[wid, pl.ds(h * HALF, HALF)], src_v)
                pltpu.sync_copy(dst_hbm.at[wid, pl.ds(h * HALF, HALF)], dst_v)
                gfire(0, 0, gsem0)
                gwait(0, 0, gsem0)
                gfire(1, 1, gsem1)
                sfire(0, 0, ssem0)
                gwait(1, 1, gsem1)
                swait(0, 0, ssem0)
                gfire(2, 0, gsem0)
                sfire(1, 1, ssem1)

                def body(g, carry):
                    a = 2 * g
                    b = a + 1
                    gwait(a, 0, gsem0)
                    swait(b - 2, 1, ssem1)
                    gfire(b, 1, gsem1)
                    sfire(a, 0, ssem0)
                    gwait(b, 1, gsem1)
                    swait(a, 0, ssem0)

                    @pl.when(g < NG - 1)
                    def _():
                        gfire(a + 2, 0, gsem0)
                    sfire(b, 1, ssem1)
                    return carry
                lax.fori_loop(1, NG, body, 0)
                swait(2 * NG - 1, 1, ssem1)
        plsc.subcore_barrier()
        pltpu.sync_copy(acc.at[pl.ds(base, RPT)], out_hbm.at[cid, pl.ds(base, RPT)])

    return k(hs, srcr, dstr)


def _tc_hs1(x, w1t, degp):
    grid = (N_NODES // BT,)

    def body(x_ref, w_ref, deg_ref, hs_ref, dinv_ref):
        dg = deg_ref[0, :, 0:1] + deg_ref[1, :, 0:1] + 1.0
        dinv = lax.rsqrt(dg)
        hs_ref[...] = jnp.dot(x_ref[...], w_ref[...],
                              preferred_element_type=jnp.float32) * dinv
        dinv_ref[...] = jnp.broadcast_to(dinv, (BT, DEGW))

    return pl.pallas_call(
        body,
        grid=grid,
        in_specs=[
            pl.BlockSpec((BT, H), lambda i: (i, 0)),
            pl.BlockSpec((H, H), lambda i: (0, 0)),
            pl.BlockSpec((NC, BT, H), lambda i: (0, i, 0)),
        ],
        out_specs=[
            pl.BlockSpec((BT, H), lambda i: (i, 0)),
            pl.BlockSpec((BT, DEGW), lambda i: (i, 0)),
        ],
        out_shape=[
            jax.ShapeDtypeStruct((N_NODES, H), jnp.float32),
            jax.ShapeDtypeStruct((N_NODES, DEGW), jnp.float32),
        ],
    )(x, w1t, degp)


def _tc_mid(aggp, hs1, dinvb, b1r, w2t):
    grid = (N_NODES // BT,)

    def body(agg_ref, hs_ref, dinv_ref, b_ref, w_ref, out_ref):
        dinv = dinv_ref[:, 0:1]
        o1 = jnp.maximum(
            dinv * (agg_ref[0] + agg_ref[1] + hs_ref[...]) + b_ref[...], 0.0)
        out_ref[...] = jnp.dot(o1, w_ref[...],
                               preferred_element_type=jnp.float32) * dinv

    return pl.pallas_call(
        body,
        grid=grid,
        in_specs=[
            pl.BlockSpec((NC, BT, H), lambda i: (0, i, 0)),
            pl.BlockSpec((BT, H), lambda i: (i, 0)),
            pl.BlockSpec((BT, DEGW), lambda i: (i, 0)),
            pl.BlockSpec((1, H), lambda i: (0, 0)),
            pl.BlockSpec((H, H), lambda i: (0, 0)),
        ],
        out_specs=pl.BlockSpec((BT, H), lambda i: (i, 0)),
        out_shape=jax.ShapeDtypeStruct((N_NODES, H), jnp.float32),
    )(aggp, hs1, dinvb, b1r, w2t)


def _tc_head(aggp, hs2, dinvb, b2r, lw1t, lb1r, lw2t, lb2r, lw3t8, lb38):
    grid = (N_NODES // BT,)

    def body(agg_ref, hs_ref, dinv_ref, b_ref, w1_ref, c1_ref, w2_ref, c2_ref,
             w3_ref, c3_ref, out_ref):
        dinv = dinv_ref[:, 0:1]
        o2 = jnp.maximum(
            dinv * (agg_ref[0] + agg_ref[1] + hs_ref[...]) + b_ref[...], 0.0)
        m1 = jnp.maximum(jnp.dot(o2, w1_ref[...],
                                 preferred_element_type=jnp.float32) + c1_ref[...], 0.0)
        m2 = jnp.maximum(jnp.dot(m1, w2_ref[...],
                                 preferred_element_type=jnp.float32) + c2_ref[...], 0.0)
        out_ref[...] = jnp.dot(m2, w3_ref[...],
                               preferred_element_type=jnp.float32) + c3_ref[...]

    return pl.pallas_call(
        body,
        grid=grid,
        in_specs=[
            pl.BlockSpec((NC, BT, H), lambda i: (0, i, 0)),
            pl.BlockSpec((BT, H), lambda i: (i, 0)),
            pl.BlockSpec((BT, DEGW), lambda i: (i, 0)),
            pl.BlockSpec((1, H), lambda i: (0, 0)),
            pl.BlockSpec((H, H), lambda i: (0, 0)),
            pl.BlockSpec((1, H), lambda i: (0, 0)),
            pl.BlockSpec((H, H), lambda i: (0, 0)),
            pl.BlockSpec((1, H), lambda i: (0, 0)),
            pl.BlockSpec((H, 8), lambda i: (0, 0)),
            pl.BlockSpec((1, 8), lambda i: (0, 0)),
        ],
        out_specs=pl.BlockSpec((BT, 8), lambda i: (i, 0)),
        out_shape=jax.ShapeDtypeStruct((N_NODES, 8), jnp.float32),
    )(aggp, hs2, dinvb, b2r, lw1t, lb1r, lw2t, lb2r, lw3t8, lb38)


def _layout_agg_edges(v, fill):
    pad = E_PAD - v.shape[0]
    chunks = jnp.concatenate(
        [v, jnp.full((pad,), fill, v.dtype)]).reshape(-1, CH)
    nf = NS * K_FAST
    cf = chunks[:nf].reshape(NS, K_FAST, CH)
    cs = jnp.concatenate(
        [chunks[nf:].reshape(NS, K_SLOW, CH),
         jnp.full((NS, NCHUNK - K_SLOW, CH), fill, v.dtype)], axis=1)
    parts = (cf, cs) if FAST_CID == 0 else (cs, cf)
    return jnp.concatenate(parts, axis=0)


def kernel(x, edge_index, W1, b1, W2, b2, lw1, lb1, lw2, lb2, lw3, lb3):
    src = edge_index[0]
    dst = edge_index[1]
    srcr = _layout_agg_edges(src, 0)
    dstr = _layout_agg_edges(dst, N_NODES)
    pad_deg = E_PAD_DEG - dst.shape[0]
    dstr_deg = jnp.concatenate(
        [dst, jnp.full((pad_deg,), N_NODES, dst.dtype)]).reshape(NW, DCHUNK, CH)

    degp = _sc_deg(dstr_deg)
    hs1, dinvb = _tc_hs1(x, W1.T, degp)
    agg1 = _sc_agg(hs1, srcr, dstr)
    hs2 = _tc_mid(agg1, hs1, dinvb, b1.reshape(1, H), W2.T)
    agg2 = _sc_agg(hs2, srcr, dstr)
    lw3t8 = jnp.concatenate([lw3.T, jnp.zeros((H, 7), jnp.float32)], axis=1)
    lb38 = jnp.concatenate([lb3, jnp.zeros((7,), jnp.float32)]).reshape(1, 8)
    y8 = _tc_head(agg2, hs2, dinvb, b2.reshape(1, H), lw1.T,
                  lb1.reshape(1, H), lw2.T, lb2.reshape(1, H), lw3t8, lb38)
    return y8[:, 0:1]


# speedup vs baseline: 9.6160x; 1.0572x over previous
import functools

import jax
import jax.numpy as jnp
from jax import lax
from jax.experimental import pallas as pl
from jax.experimental.pallas import tpu as pltpu
from jax.experimental.pallas import tpu_sc as plsc

N_NODES = 10000
H = 128
NC = 2
NS = 16
NW = NC * NS
CH = 128
ACC_ROWS = 10240
RPT = ACC_ROWS // NS
DEGW = 16
BT = 1000

HALF = 40
K_FAST = 3 * HALF
K_SLOW = 1 * HALF
FAST_CID = 1
NCHUNK = K_FAST
E_PAD = NS * (K_FAST + K_SLOW) * CH

DCHUNK = 80
DHALF = DCHUNK // 2
E_PAD_DEG = NW * DCHUNK * CH


def _sc_deg(dstr):
    mesh = plsc.VectorSubcoreMesh(core_axis_name="c", subcore_axis_name="s")

    @functools.partial(
        pl.kernel,
        out_type=jax.ShapeDtypeStruct((NC, ACC_ROWS, H), jnp.float32),
        mesh=mesh,
        scratch_types=[
            pltpu.VMEM((DHALF, CH), jnp.int32),
            pltpu.VMEM((CH, H), jnp.float32),
            pltpu.VMEM_SHARED((ACC_ROWS, H), jnp.float32),
            pltpu.SemaphoreType.DMA,
        ],
    )
    def k(dst_hbm, out_hbm, dst_v, ones_v, acc, sem):
        cid = lax.axis_index("c")
        sid = lax.axis_index("s")
        wid = cid * NS + sid

        def fill(val):
            def body(i, carry):
                for j in range(H // 16):
                    ones_v[i, pl.ds(j * 16, 16)] = jnp.full((16,), val, jnp.float32)
                return carry
            lax.fori_loop(0, CH, body, 0)

        fill(0.0)
        base = sid * RPT
        for j in range(RPT // CH):
            pltpu.sync_copy(ones_v, acc.at[pl.ds(base + j * CH, CH)])
        fill(1.0)
        plsc.subcore_barrier()

        def body(g, carry):
            ds_ = [pltpu.async_copy(ones_v, acc.at[dst_v.at[g * 2 + b]], sem,
                                    add=True)
                   for b in range(2)]
            for d in ds_:
                d.wait()
            return carry

        for h in range(DCHUNK // DHALF):
            pltpu.sync_copy(dst_hbm.at[wid, pl.ds(h * DHALF, DHALF)], dst_v)
            lax.fori_loop(0, DHALF // 2, body, 0)
        plsc.subcore_barrier()
        pltpu.sync_copy(acc.at[pl.ds(base, RPT)], out_hbm.at[cid, pl.ds(base, RPT)])

    return k(dstr)


def _sc_agg(hs, srcr, dstr):
    mesh = plsc.VectorSubcoreMesh(core_axis_name="c", subcore_axis_name="s")

    @functools.partial(
        pl.kernel,
        out_type=jax.ShapeDtypeStruct((NC, ACC_ROWS, H), jnp.float32),
        mesh=mesh,
        scratch_types=[
            pltpu.VMEM((HALF, CH), jnp.int32),
            pltpu.VMEM((HALF, CH), jnp.int32),
            pltpu.VMEM((2, CH, H), jnp.float32),
            pltpu.VMEM_SHARED((ACC_ROWS, H), jnp.float32),
            pltpu.SemaphoreType.DMA,
            pltpu.SemaphoreType.DMA,
            pltpu.SemaphoreType.DMA,
            pltpu.SemaphoreType.DMA,
        ],
    )
    def k(hs_hbm, src_hbm, dst_hbm, out_hbm, src_v, dst_v, rows_v, acc,
          gsem0, gsem1, ssem0, ssem1):
        cid = lax.axis_index("c")
        sid = lax.axis_index("s")
        wid = cid * NS + sid
        nreal = jnp.where(cid == FAST_CID, K_FAST, K_SLOW)

        def zbody(i, carry):
            for j in range(H // 16):
                rows_v[0, i, pl.ds(j * 16, 16)] = jnp.zeros((16,), jnp.float32)
            return carry
        lax.fori_loop(0, CH, zbody, 0)
        base = sid * RPT
        for j in range(RPT // CH):
            pltpu.sync_copy(rows_v.at[0], acc.at[pl.ds(base + j * CH, CH)])
        plsc.subcore_barrier()

        def gfire(c, slot, sem):
            pltpu.async_copy(hs_hbm.at[src_v.at[c]], rows_v.at[slot], sem)

        def gwait(c, slot, sem):
            pltpu.make_async_copy(hs_hbm.at[src_v.at[c]], rows_v.at[slot],
                                  sem).wait()

        def sfire(c, slot, sem):
            pltpu.async_copy(rows_v.at[slot], acc.at[dst_v.at[c]], sem,
                             add=True)

        def swait(c, slot, sem):
            pltpu.make_async_copy(rows_v.at[slot], acc.at[dst_v.at[c]],
                                  sem).wait()

        NG = HALF // 2
        for h in range(NCHUNK // HALF):

            @pl.when(h * HALF < nreal)
            def _stage(h=h):
                pltpu.sync_copy(src_hbm.at[wid, pl.ds(h * HALF, HALF)], src_v)
                pltpu.sync_copy(dst_hbm.at[wid, pl.ds(h * HALF, HALF)], dst_v)
                gfire(0, 0, gsem0)
                gwait(0, 0, gsem0)
                gfire(1, 1, gsem1)
                sfire(0, 0, ssem0)
                gwait(1, 1, gsem1)
                swait(0, 0, ssem0)
                gfire(2, 0, gsem0)
                sfire(1, 1, ssem1)

                def body(g, carry):
                    a = 2 * g
                    b = a + 1
                    gwait(a, 0, gsem0)
                    swait(b - 2, 1, ssem1)
                    gfire(b, 1, gsem1)
                    sfire(a, 0, ssem0)
                    gwait(b, 1, gsem1)
                    swait(a, 0, ssem0)

                    @pl.when(g < NG - 1)
                    def _():
                        gfire(a + 2, 0, gsem0)
                    sfire(b, 1, ssem1)
                    return carry
                lax.fori_loop(1, NG, body, 0)
                swait(2 * NG - 1, 1, ssem1)
        plsc.subcore_barrier()
        pltpu.sync_copy(acc.at[pl.ds(base, RPT)], out_hbm.at[cid, pl.ds(base, RPT)])

    return k(hs, srcr, dstr)


def _tc_hs1(x, w1t, degp):
    grid = (N_NODES // BT,)

    def body(x_ref, w_ref, deg_ref, hs_ref, dinv_ref):
        dg = deg_ref[0, :, 0:1] + deg_ref[1, :, 0:1] + 1.0
        dinv = lax.rsqrt(dg)
        hs_ref[...] = jnp.dot(x_ref[...], w_ref[...],
                              preferred_element_type=jnp.float32) * dinv
        dinv_ref[...] = jnp.broadcast_to(dinv, (BT, DEGW))

    return pl.pallas_call(
        body,
        grid=grid,
        in_specs=[
            pl.BlockSpec((BT, H), lambda i: (i, 0)),
            pl.BlockSpec((H, H), lambda i: (0, 0)),
            pl.BlockSpec((NC, BT, H), lambda i: (0, i, 0)),
        ],
        out_specs=[
            pl.BlockSpec((BT, H), lambda i: (i, 0)),
            pl.BlockSpec((BT, DEGW), lambda i: (i, 0)),
        ],
        out_shape=[
            jax.ShapeDtypeStruct((N_NODES, H), jnp.float32),
            jax.ShapeDtypeStruct((N_NODES, DEGW), jnp.float32),
        ],
    )(x, w1t, degp)


def _tc_mid(aggp, hs1, dinvb, b1r, w2t):
    grid = (N_NODES // BT,)

    def body(agg_ref, hs_ref, dinv_ref, b_ref, w_ref, out_ref):
        dinv = dinv_ref[:, 0:1]
        o1 = jnp.maximum(
            dinv * (agg_ref[0] + agg_ref[1] + hs_ref[...]) + b_ref[...], 0.0)
        out_ref[...] = jnp.dot(o1, w_ref[...],
                               preferred_element_type=jnp.float32) * dinv

    return pl.pallas_call(
        body,
        grid=grid,
        in_specs=[
            pl.BlockSpec((NC, BT, H), lambda i: (0, i, 0)),
            pl.BlockSpec((BT, H), lambda i: (i, 0)),
            pl.BlockSpec((BT, DEGW), lambda i: (i, 0)),
            pl.BlockSpec((1, H), lambda i: (0, 0)),
            pl.BlockSpec((H, H), lambda i: (0, 0)),
        ],
        out_specs=pl.BlockSpec((BT, H), lambda i: (i, 0)),
        out_shape=jax.ShapeDtypeStruct((N_NODES, H), jnp.float32),
    )(aggp, hs1, dinvb, b1r, w2t)


def _tc_head(aggp, hs2, dinvb, b2r, lw1t, lb1r, lw2t, lb2r, lw3t8, lb38):
    grid = (N_NODES // BT,)

    def body(agg_ref, hs_ref, dinv_ref, b_ref, w1_ref, c1_ref, w2_ref, c2_ref,
             w3_ref, c3_ref, out_ref):
        dinv = dinv_ref[:, 0:1]
        o2 = jnp.maximum(
            dinv * (agg_ref[0] + agg_ref[1] + hs_ref[...]) + b_ref[...], 0.0)
        m1 = jnp.maximum(jnp.dot(o2, w1_ref[...],
                                 preferred_element_type=jnp.float32) + c1_ref[...], 0.0)
        m2 = jnp.maximum(jnp.dot(m1, w2_ref[...],
                                 preferred_element_type=jnp.float32) + c2_ref[...], 0.0)
        out_ref[...] = jnp.dot(m2, w3_ref[...],
                               preferred_element_type=jnp.float32) + c3_ref[...]

    return pl.pallas_call(
        body,
        grid=grid,
        in_specs=[
            pl.BlockSpec((NC, BT, H), lambda i: (0, i, 0)),
            pl.BlockSpec((BT, H), lambda i: (i, 0)),
            pl.BlockSpec((BT, DEGW), lambda i: (i, 0)),
            pl.BlockSpec((1, H), lambda i: (0, 0)),
            pl.BlockSpec((H, H), lambda i: (0, 0)),
            pl.BlockSpec((1, H), lambda i: (0, 0)),
            pl.BlockSpec((H, H), lambda i: (0, 0)),
            pl.BlockSpec((1, H), lambda i: (0, 0)),
            pl.BlockSpec((H, 8), lambda i: (0, 0)),
            pl.BlockSpec((1, 8), lambda i: (0, 0)),
        ],
        out_specs=pl.BlockSpec((BT, 8), lambda i: (i, 0)),
        out_shape=jax.ShapeDtypeStruct((N_NODES, 8), jnp.float32),
    )(aggp, hs2, dinvb, b2r, lw1t, lb1r, lw2t, lb2r, lw3t8, lb38)


def _layout_agg_edges(v, fill):
    pad = E_PAD - v.shape[0]
    chunks = jnp.concatenate(
        [v, jnp.full((pad,), fill, v.dtype)]).reshape(-1, CH)
    nf = NS * K_FAST
    cf = chunks[:nf].reshape(NS, K_FAST, CH)
    cs = jnp.concatenate(
        [chunks[nf:].reshape(NS, K_SLOW, CH),
         jnp.full((NS, NCHUNK - K_SLOW, CH), fill, v.dtype)], axis=1)
    parts = (cf, cs) if FAST_CID == 0 else (cs, cf)
    return jnp.concatenate(parts, axis=0)


def kernel(x, edge_index, W1, b1, W2, b2, lw1, lb1, lw2, lb2, lw3, lb3):
    src = edge_index[0]
    dst = edge_index[1]
    srcr = _layout_agg_edges(src, 0)
    dstr = _layout_agg_edges(dst, N_NODES)
    pad_deg = E_PAD_DEG - dst.shape[0]
    dstr_deg = jnp.concatenate(
        [dst, jnp.full((pad_deg,), N_NODES, dst.dtype)]).reshape(NW, DCHUNK, CH)

    degp = _sc_deg(dstr_deg)
    hs1, dinvb = _tc_hs1(x, W1.T, degp)
    agg1 = _sc_agg(hs1, srcr, dstr)
    hs2 = _tc_mid(agg1, hs1, dinvb, b1.reshape(1, H), W2.T)
    agg2 = _sc_agg(hs2, srcr, dstr)
    lw3t8 = jnp.concatenate([lw3.T, jnp.zeros((H, 7), jnp.float32)], axis=1)
    lb38 = jnp.concatenate([lb3, jnp.zeros((7,), jnp.float32)]).reshape(1, 8)
    y8 = _tc_head(agg2, hs2, dinvb, b2.reshape(1, H), lw1.T,
                  lb1.reshape(1, H), lw2.T, lb2.reshape(1, H), lw3t8, lb38)
    return y8[:, 0:1]


# speedup vs baseline: 10.4353x; 1.0852x over previous
import functools

import jax
import jax.numpy as jnp
from jax import lax
from jax.experimental import pallas as pl
from jax.experimental.pallas import tpu as pltpu
from jax.experimental.pallas import tpu_sc as plsc

N_NODES = 10000
H = 128
NC = 2
NS = 16
NW = NC * NS
CH = 128
ACC_ROWS = 10240
RPT = ACC_ROWS // NS
DEGW = 16
BT = 1000

NCHUNK = 79
E_PAD = NW * NCHUNK * CH

DCHUNK = 80
DHALF = DCHUNK // 2
E_PAD_DEG = NW * DCHUNK * CH


def _sc_deg(dstr):
    mesh = plsc.VectorSubcoreMesh(core_axis_name="c", subcore_axis_name="s")

    @functools.partial(
        pl.kernel,
        out_type=jax.ShapeDtypeStruct((NC, ACC_ROWS, H), jnp.float32),
        mesh=mesh,
        scratch_types=[
            pltpu.VMEM((DHALF, CH), jnp.int32),
            pltpu.VMEM((CH, H), jnp.float32),
            pltpu.VMEM_SHARED((ACC_ROWS, H), jnp.float32),
            pltpu.SemaphoreType.DMA,
        ],
    )
    def k(dst_hbm, out_hbm, dst_v, ones_v, acc, sem):
        cid = lax.axis_index("c")
        sid = lax.axis_index("s")
        wid = cid * NS + sid

        def fill(val):
            def body(i, carry):
                for j in range(H // 16):
                    ones_v[i, pl.ds(j * 16, 16)] = jnp.full((16,), val, jnp.float32)
                return carry
            lax.fori_loop(0, CH, body, 0)

        fill(0.0)
        base = sid * RPT
        for j in range(RPT // CH):
            pltpu.sync_copy(ones_v, acc.at[pl.ds(base + j * CH, CH)])
        fill(1.0)
        plsc.subcore_barrier()

        def body(g, carry):
            ds_ = [pltpu.async_copy(ones_v, acc.at[dst_v.at[g * 2 + b]], sem,
                                    add=True)
                   for b in range(2)]
            for d in ds_:
                d.wait()
            return carry

        for h in range(DCHUNK // DHALF):
            pltpu.sync_copy(dst_hbm.at[wid, pl.ds(h * DHALF, DHALF)], dst_v)
            lax.fori_loop(0, DHALF // 2, body, 0)
        plsc.subcore_barrier()
        pltpu.sync_copy(acc.at[pl.ds(base, RPT)], out_hbm.at[cid, pl.ds(base, RPT)])

    return k(dstr)


def _sc_agg(hs, srcr, dstr):
    mesh = plsc.VectorSubcoreMesh(core_axis_name="c", subcore_axis_name="s")

    @functools.partial(
        pl.kernel,
        out_type=jax.ShapeDtypeStruct((NC, ACC_ROWS, H), jnp.float32),
        mesh=mesh,
        scratch_types=[
            pltpu.VMEM((CH,), jnp.int32),
            pltpu.VMEM((CH,), jnp.int32),
            pltpu.VMEM((CH, H), jnp.float32),
            pltpu.VMEM_SHARED((ACC_ROWS, H), jnp.float32),
        ],
    )
    def k(hs_hbm, src_hbm, dst_hbm, out_hbm, src_v, dst_v, rows_v, acc):
        cid = lax.axis_index("c")
        sid = lax.axis_index("s")
        wid = cid * NS + sid

        def zbody(i, carry):
            for j in range(H // 16):
                rows_v[i, pl.ds(j * 16, 16)] = jnp.zeros((16,), jnp.float32)
            return carry
        lax.fori_loop(0, CH, zbody, 0)
        base = sid * RPT
        for j in range(RPT // CH):
            pltpu.sync_copy(rows_v, acc.at[pl.ds(base + j * CH, CH)])
        plsc.subcore_barrier()

        def body(i, carry):
            pltpu.sync_copy(src_hbm.at[wid, i], src_v)
            pltpu.sync_copy(dst_hbm.at[wid, i], dst_v)
            pltpu.sync_copy(hs_hbm.at[src_v], rows_v)
            pltpu.sync_copy(rows_v, acc.at[dst_v], add=True)
            return carry
        lax.fori_loop(0, NCHUNK, body, 0)
        plsc.subcore_barrier()
        pltpu.sync_copy(acc.at[pl.ds(base, RPT)], out_hbm.at[cid, pl.ds(base, RPT)])

    return k(hs, srcr, dstr)


def _tc_hs1(x, w1t, degp):
    grid = (N_NODES // BT,)

    def body(x_ref, w_ref, deg_ref, hs_ref, dinv_ref):
        dg = deg_ref[0, :, 0:1] + deg_ref[1, :, 0:1] + 1.0
        dinv = lax.rsqrt(dg)
        hs_ref[...] = jnp.dot(x_ref[...], w_ref[...],
                              preferred_element_type=jnp.float32) * dinv
        dinv_ref[...] = jnp.broadcast_to(dinv, (BT, DEGW))

    return pl.pallas_call(
        body,
        grid=grid,
        in_specs=[
            pl.BlockSpec((BT, H), lambda i: (i, 0)),
            pl.BlockSpec((H, H), lambda i: (0, 0)),
            pl.BlockSpec((NC, BT, H), lambda i: (0, i, 0)),
        ],
        out_specs=[
            pl.BlockSpec((BT, H), lambda i: (i, 0)),
            pl.BlockSpec((BT, DEGW), lambda i: (i, 0)),
        ],
        out_shape=[
            jax.ShapeDtypeStruct((N_NODES, H), jnp.float32),
            jax.ShapeDtypeStruct((N_NODES, DEGW), jnp.float32),
        ],
    )(x, w1t, degp)


def _tc_mid(aggp, hs1, dinvb, b1r, w2t):
    grid = (N_NODES // BT,)

    def body(agg_ref, hs_ref, dinv_ref, b_ref, w_ref, out_ref):
        dinv = dinv_ref[:, 0:1]
        o1 = jnp.maximum(
            dinv * (agg_ref[0] + agg_ref[1] + hs_ref[...]) + b_ref[...], 0.0)
        out_ref[...] = jnp.dot(o1, w_ref[...],
                               preferred_element_type=jnp.float32) * dinv

    return pl.pallas_call(
        body,
        grid=grid,
        in_specs=[
            pl.BlockSpec((NC, BT, H), lambda i: (0, i, 0)),
            pl.BlockSpec((BT, H), lambda i: (i, 0)),
            pl.BlockSpec((BT, DEGW), lambda i: (i, 0)),
            pl.BlockSpec((1, H), lambda i: (0, 0)),
            pl.BlockSpec((H, H), lambda i: (0, 0)),
        ],
        out_specs=pl.BlockSpec((BT, H), lambda i: (i, 0)),
        out_shape=jax.ShapeDtypeStruct((N_NODES, H), jnp.float32),
    )(aggp, hs1, dinvb, b1r, w2t)


def _tc_head(aggp, hs2, dinvb, b2r, lw1t, lb1r, lw2t, lb2r, lw3t8, lb38):
    grid = (N_NODES // BT,)

    def body(agg_ref, hs_ref, dinv_ref, b_ref, w1_ref, c1_ref, w2_ref, c2_ref,
             w3_ref, c3_ref, out_ref):
        dinv = dinv_ref[:, 0:1]
        o2 = jnp.maximum(
            dinv * (agg_ref[0] + agg_ref[1] + hs_ref[...]) + b_ref[...], 0.0)
        m1 = jnp.maximum(jnp.dot(o2, w1_ref[...],
                                 preferred_element_type=jnp.float32) + c1_ref[...], 0.0)
        m2 = jnp.maximum(jnp.dot(m1, w2_ref[...],
                                 preferred_element_type=jnp.float32) + c2_ref[...], 0.0)
        out_ref[...] = jnp.dot(m2, w3_ref[...],
                               preferred_element_type=jnp.float32) + c3_ref[...]

    return pl.pallas_call(
        body,
        grid=grid,
        in_specs=[
            pl.BlockSpec((NC, BT, H), lambda i: (0, i, 0)),
            pl.BlockSpec((BT, H), lambda i: (i, 0)),
            pl.BlockSpec((BT, DEGW), lambda i: (i, 0)),
            pl.BlockSpec((1, H), lambda i: (0, 0)),
            pl.BlockSpec((H, H), lambda i: (0, 0)),
            pl.BlockSpec((1, H), lambda i: (0, 0)),
            pl.BlockSpec((H, H), lambda i: (0, 0)),
            pl.BlockSpec((1, H), lambda i: (0, 0)),
            pl.BlockSpec((H, 8), lambda i: (0, 0)),
            pl.BlockSpec((1, 8), lambda i: (0, 0)),
        ],
        out_specs=pl.BlockSpec((BT, 8), lambda i: (i, 0)),
        out_shape=jax.ShapeDtypeStruct((N_NODES, 8), jnp.float32),
    )(aggp, hs2, dinvb, b2r, lw1t, lb1r, lw2t, lb2r, lw3t8, lb38)


def _layout_agg_edges(v, fill):
    pad = E_PAD - v.shape[0]
    return jnp.concatenate(
        [v, jnp.full((pad,), fill, v.dtype)]).reshape(NW, NCHUNK, CH)


def kernel(x, edge_index, W1, b1, W2, b2, lw1, lb1, lw2, lb2, lw3, lb3):
    src = edge_index[0]
    dst = edge_index[1]
    srcr = _layout_agg_edges(src, 0)
    dstr = _layout_agg_edges(dst, N_NODES)
    pad_deg = E_PAD_DEG - dst.shape[0]
    dstr_deg = jnp.concatenate(
        [dst, jnp.full((pad_deg,), N_NODES, dst.dtype)]).reshape(NW, DCHUNK, CH)

    degp = _sc_deg(dstr_deg)
    hs1, dinvb = _tc_hs1(x, W1.T, degp)
    agg1 = _sc_agg(hs1, srcr, dstr)
    hs2 = _tc_mid(agg1, hs1, dinvb, b1.reshape(1, H), W2.T)
    agg2 = _sc_agg(hs2, srcr, dstr)
    lw3t8 = jnp.concatenate([lw3.T, jnp.zeros((H, 7), jnp.float32)], axis=1)
    lb38 = jnp.concatenate([lb3, jnp.zeros((7,), jnp.float32)]).reshape(1, 8)
    y8 = _tc_head(agg2, hs2, dinvb, b2.reshape(1, H), lw1.T,
                  lb1.reshape(1, H), lw2.T, lb2.reshape(1, H), lw3t8, lb38)
    return y8[:, 0:1]


# speedup vs baseline: 10.6594x; 1.0215x over previous
import functools

import jax
import jax.numpy as jnp
from jax import lax
from jax.experimental import pallas as pl
from jax.experimental.pallas import tpu as pltpu
from jax.experimental.pallas import tpu_sc as plsc

N_NODES = 10000
H = 128
NC = 2
NS = 16
NW = NC * NS
CH = 128
ACC_ROWS = 10240
RPT = ACC_ROWS // NS
DEGW = 16
BT = 1000

NCHUNK = 79
E_PAD = NW * NCHUNK * CH


def _sc_deg(dstr):
    mesh = plsc.VectorSubcoreMesh(core_axis_name="c", subcore_axis_name="s")

    @functools.partial(
        pl.kernel,
        out_type=jax.ShapeDtypeStruct((NC, ACC_ROWS, H), jnp.float32),
        mesh=mesh,
        scratch_types=[
            pltpu.VMEM((CH,), jnp.int32),
            pltpu.VMEM((CH, H), jnp.float32),
            pltpu.VMEM_SHARED((ACC_ROWS, H), jnp.float32),
        ],
    )
    def k(dst_hbm, out_hbm, dst_v, ones_v, acc):
        cid = lax.axis_index("c")
        sid = lax.axis_index("s")
        wid = cid * NS + sid

        def fill(val):
            def body(i, carry):
                for j in range(H // 16):
                    ones_v[i, pl.ds(j * 16, 16)] = jnp.full((16,), val, jnp.float32)
                return carry
            lax.fori_loop(0, CH, body, 0)

        fill(0.0)
        base = sid * RPT
        for j in range(RPT // CH):
            pltpu.sync_copy(ones_v, acc.at[pl.ds(base + j * CH, CH)])
        fill(1.0)
        plsc.subcore_barrier()

        def body(i, carry):
            pltpu.sync_copy(dst_hbm.at[wid, i], dst_v)
            pltpu.sync_copy(ones_v, acc.at[dst_v], add=True)
            return carry
        lax.fori_loop(0, NCHUNK, body, 0)
        plsc.subcore_barrier()
        pltpu.sync_copy(acc.at[pl.ds(base, RPT)], out_hbm.at[cid, pl.ds(base, RPT)])

    return k(dstr)


def _sc_agg(hs, srcr, dstr):
    mesh = plsc.VectorSubcoreMesh(core_axis_name="c", subcore_axis_name="s")

    @functools.partial(
        pl.kernel,
        out_type=jax.ShapeDtypeStruct((NC, ACC_ROWS, H), jnp.float32),
        mesh=mesh,
        scratch_types=[
            pltpu.VMEM((CH,), jnp.int32),
            pltpu.VMEM((CH,), jnp.int32),
            pltpu.VMEM((CH, H), jnp.float32),
            pltpu.VMEM_SHARED((ACC_ROWS, H), jnp.float32),
        ],
    )
    def k(hs_hbm, src_hbm, dst_hbm, out_hbm, src_v, dst_v, rows_v, acc):
        cid = lax.axis_index("c")
        sid = lax.axis_index("s")
        wid = cid * NS + sid

        def zbody(i, carry):
            for j in range(H // 16):
                rows_v[i, pl.ds(j * 16, 16)] = jnp.zeros((16,), jnp.float32)
            return carry
        lax.fori_loop(0, CH, zbody, 0)
        base = sid * RPT
        for j in range(RPT // CH):
            pltpu.sync_copy(rows_v, acc.at[pl.ds(base + j * CH, CH)])
        plsc.subcore_barrier()

        def body(i, carry):
            pltpu.sync_copy(src_hbm.at[wid, i], src_v)
            pltpu.sync_copy(dst_hbm.at[wid, i], dst_v)
            pltpu.sync_copy(hs_hbm.at[src_v], rows_v)
            pltpu.sync_copy(rows_v, acc.at[dst_v], add=True)
            return carry
        lax.fori_loop(0, NCHUNK, body, 0)
        plsc.subcore_barrier()
        pltpu.sync_copy(acc.at[pl.ds(base, RPT)], out_hbm.at[cid, pl.ds(base, RPT)])

    return k(hs, srcr, dstr)


def _tc_hs1(x, w1t, degp):
    grid = (N_NODES // BT,)

    def body(x_ref, w_ref, deg_ref, hs_ref, dinv_ref):
        dg = deg_ref[0, :, 0:1] + deg_ref[1, :, 0:1] + 1.0
        dinv = lax.rsqrt(dg)
        hs_ref[...] = jnp.dot(x_ref[...], w_ref[...],
                              preferred_element_type=jnp.float32) * dinv
        dinv_ref[...] = jnp.broadcast_to(dinv, (BT, DEGW))

    return pl.pallas_call(
        body,
        grid=grid,
        in_specs=[
            pl.BlockSpec((BT, H), lambda i: (i, 0)),
            pl.BlockSpec((H, H), lambda i: (0, 0)),
            pl.BlockSpec((NC, BT, H), lambda i: (0, i, 0)),
        ],
        out_specs=[
            pl.BlockSpec((BT, H), lambda i: (i, 0)),
            pl.BlockSpec((BT, DEGW), lambda i: (i, 0)),
        ],
        out_shape=[
            jax.ShapeDtypeStruct((N_NODES, H), jnp.float32),
            jax.ShapeDtypeStruct((N_NODES, DEGW), jnp.float32),
        ],
    )(x, w1t, degp)


def _tc_mid(aggp, hs1, dinvb, b1r, w2t):
    grid = (N_NODES // BT,)

    def body(agg_ref, hs_ref, dinv_ref, b_ref, w_ref, out_ref):
        dinv = dinv_ref[:, 0:1]
        o1 = jnp.maximum(
            dinv * (agg_ref[0] + agg_ref[1] + hs_ref[...]) + b_ref[...], 0.0)
        out_ref[...] = jnp.dot(o1, w_ref[...],
                               preferred_element_type=jnp.float32) * dinv

    return pl.pallas_call(
        body,
        grid=grid,
        in_specs=[
            pl.BlockSpec((NC, BT, H), lambda i: (0, i, 0)),
            pl.BlockSpec((BT, H), lambda i: (i, 0)),
            pl.BlockSpec((BT, DEGW), lambda i: (i, 0)),
            pl.BlockSpec((1, H), lambda i: (0, 0)),
            pl.BlockSpec((H, H), lambda i: (0, 0)),
        ],
        out_specs=pl.BlockSpec((BT, H), lambda i: (i, 0)),
        out_shape=jax.ShapeDtypeStruct((N_NODES, H), jnp.float32),
    )(aggp, hs1, dinvb, b1r, w2t)


def _tc_head(aggp, hs2, dinvb, b2r, lw1t, lb1r, lw2t, lb2r, lw3t8, lb38):
    grid = (N_NODES // BT,)

    def body(agg_ref, hs_ref, dinv_ref, b_ref, w1_ref, c1_ref, w2_ref, c2_ref,
             w3_ref, c3_ref, out_ref):
        dinv = dinv_ref[:, 0:1]
        o2 = jnp.maximum(
            dinv * (agg_ref[0] + agg_ref[1] + hs_ref[...]) + b_ref[...], 0.0)
        m1 = jnp.maximum(jnp.dot(o2, w1_ref[...],
                                 preferred_element_type=jnp.float32) + c1_ref[...], 0.0)
        m2 = jnp.maximum(jnp.dot(m1, w2_ref[...],
                                 preferred_element_type=jnp.float32) + c2_ref[...], 0.0)
        out_ref[...] = jnp.dot(m2, w3_ref[...],
                               preferred_element_type=jnp.float32) + c3_ref[...]

    return pl.pallas_call(
        body,
        grid=grid,
        in_specs=[
            pl.BlockSpec((NC, BT, H), lambda i: (0, i, 0)),
            pl.BlockSpec((BT, H), lambda i: (i, 0)),
            pl.BlockSpec((BT, DEGW), lambda i: (i, 0)),
            pl.BlockSpec((1, H), lambda i: (0, 0)),
            pl.BlockSpec((H, H), lambda i: (0, 0)),
            pl.BlockSpec((1, H), lambda i: (0, 0)),
            pl.BlockSpec((H, H), lambda i: (0, 0)),
            pl.BlockSpec((1, H), lambda i: (0, 0)),
            pl.BlockSpec((H, 8), lambda i: (0, 0)),
            pl.BlockSpec((1, 8), lambda i: (0, 0)),
        ],
        out_specs=pl.BlockSpec((BT, 8), lambda i: (i, 0)),
        out_shape=jax.ShapeDtypeStruct((N_NODES, 8), jnp.float32),
    )(aggp, hs2, dinvb, b2r, lw1t, lb1r, lw2t, lb2r, lw3t8, lb38)


def _layout_agg_edges(v, fill):
    pad = E_PAD - v.shape[0]
    return jnp.concatenate(
        [v, jnp.full((pad,), fill, v.dtype)]).reshape(NW, NCHUNK, CH)


def kernel(x, edge_index, W1, b1, W2, b2, lw1, lb1, lw2, lb2, lw3, lb3):
    src = edge_index[0]
    dst = edge_index[1]
    srcr = _layout_agg_edges(src, 0)
    dstr = _layout_agg_edges(dst, N_NODES)

    degp = _sc_deg(dstr)
    hs1, dinvb = _tc_hs1(x, W1.T, degp)
    agg1 = _sc_agg(hs1, srcr, dstr)
    hs2 = _tc_mid(agg1, hs1, dinvb, b1.reshape(1, H), W2.T)
    agg2 = _sc_agg(hs2, srcr, dstr)
    lw3t8 = jnp.concatenate([lw3.T, jnp.zeros((H, 7), jnp.float32)], axis=1)
    lb38 = jnp.concatenate([lb3, jnp.zeros((7,), jnp.float32)]).reshape(1, 8)
    y8 = _tc_head(agg2, hs2, dinvb, b2.reshape(1, H), lw1.T,
                  lb1.reshape(1, H), lw2.T, lb2.reshape(1, H), lw3t8, lb38)
    return y8[:, 0:1]
